# Initial kernel scaffold; baseline (speedup 1.0000x reference)
import jax
import jax.numpy as jnp
from jax.experimental import pallas as pl


def kernel(seq_inputs, e1_pos_inputs, e2_pos_inputs, we, wpe):
    raise NotImplementedError("write your pallas kernel here")


# speedup vs baseline: 4.3258x; 4.3258x over previous
import jax
import jax.numpy as jnp
from jax import lax
from jax.experimental import pallas as pl
from jax.experimental.pallas import tpu as pltpu
from jax.experimental.pallas import tpu_sc as plsc

_PAD = 2
_DW = 64
_DP = 16
_DOUT = _DW + 2 * _DP
_W = 128
_NC = 2
_NS = 16
_NW = _NC * _NS


def _gather_concat(we, wpe, si, e1, e2, rows):
    rpw = rows // _NW
    nwin = rpw // _W
    mesh = plsc.VectorSubcoreMesh(core_axis_name="core", subcore_axis_name="subcore")

    @pl.kernel(
        out_type=jax.ShapeDtypeStruct((rows, _DOUT), jnp.float32),
        mesh=mesh,
        compiler_params=pltpu.CompilerParams(use_tc_tiling_on_sc=False),
        scratch_types=[
            pltpu.VMEM((nwin, _W), jnp.int32),
            pltpu.VMEM((nwin, _W), jnp.int32),
            pltpu.VMEM((nwin, _W), jnp.int32),
            pltpu.VMEM((_W, _DW), jnp.float32),
            pltpu.VMEM((_W, _DP), jnp.float32),
            pltpu.VMEM((_W, _DP), jnp.float32),
            pltpu.VMEM_SHARED((_NS, _W, _DOUT), jnp.float32),
            pltpu.SemaphoreType.DMA,
            pltpu.SemaphoreType.DMA,
        ],
    )
    def k(we_hbm, wpe_hbm, si_hbm, e1_hbm, e2_hbm, o_hbm,
          isi, ie1, ie2, bwe, be1, be2, shb, gsem, wsem):
        sid = lax.axis_index("subcore")
        wid = lax.axis_index("core") * _NS + sid
        base0 = wid * rpw
        mysh = shb.at[sid]
        c0 = pltpu.async_copy(si_hbm.at[wid], isi, gsem)
        c1 = pltpu.async_copy(e1_hbm.at[wid], ie1, gsem)
        c2 = pltpu.async_copy(e2_hbm.at[wid], ie2, gsem)
        c0.wait(); c1.wait(); c2.wait()

        @pl.loop(0, nwin)
        def _(j):
            base = base0 + j * _W
            g0 = pltpu.async_copy(we_hbm.at[isi.at[j]], bwe, gsem)
            g1 = pltpu.async_copy(wpe_hbm.at[ie1.at[j]], be1, gsem)
            g2 = pltpu.async_copy(wpe_hbm.at[ie2.at[j]], be2, gsem)
            g0.wait(); g1.wait(); g2.wait()
            a0 = pltpu.async_copy(bwe, mysh.at[:, pl.ds(0, _DW)], wsem)
            a1 = pltpu.async_copy(be1, mysh.at[:, pl.ds(_DW, _DP)], wsem)
            a2 = pltpu.async_copy(be2, mysh.at[:, pl.ds(_DW + _DP, _DP)], wsem)
            a0.wait(); a1.wait(); a2.wait()
            pltpu.sync_copy(mysh, o_hbm.at[pl.ds(base, _W)])

    return k(we, wpe, si, e1, e2)


def kernel(seq_inputs, e1_pos_inputs, e2_pos_inputs, we, wpe):
    b, t = seq_inputs.shape
    tp = t + 2 * _PAD
    rows = b * tp
    rpw = rows // _NW
    nwin = rpw // _W

    si = seq_inputs.astype(jnp.int32)
    e1 = e1_pos_inputs.astype(jnp.int32)
    e2 = e2_pos_inputs.astype(jnp.int32)

    zpad = jnp.zeros((b, _PAD), jnp.int32)
    si_p = jnp.concatenate([zpad, si, zpad], axis=1)

    def edge_pad(x):
        head = jnp.repeat(x[:, :1], _PAD, axis=1)
        tail = jnp.repeat(x[:, -1:], _PAD, axis=1)
        return jnp.concatenate([head, x, tail], axis=1)

    e1_p = edge_pad(e1)
    e2_p = edge_pad(e2)

    out = _gather_concat(
        we, wpe,
        si_p.reshape(_NW, nwin, _W),
        e1_p.reshape(_NW, nwin, _W),
        e2_p.reshape(_NW, nwin, _W),
        rows,
    )
    return out.reshape(b, tp, _DOUT)
